# Initial kernel scaffold; baseline (speedup 1.0000x reference)
import jax
import jax.numpy as jnp
from jax.experimental import pallas as pl


def kernel(x, gate_w, gate_b, fc1_w, fc2_w):
    raise NotImplementedError("write your pallas kernel here")


# speedup vs baseline: 2.0260x; 2.0260x over previous
import jax
import jax.numpy as jnp
from jax.experimental import pallas as pl
from jax.experimental.pallas import tpu as pltpu

H = 2048
INNER = 1024
E = 8
T = 2048
TI = 256
NI = INNER // TI


def _router_kernel(x_ref, gate_w_ref, gate_b_ref, gate_ref):
    logits = jax.lax.dot_general(
        x_ref[...], gate_w_ref[...], (((1,), (1,)), ((), ())),
        preferred_element_type=jnp.float32,
    ) + gate_b_ref[...]
    col = jax.lax.broadcasted_iota(jnp.int32, (T, E), 1)
    m1 = jnp.max(logits, axis=1, keepdims=True)
    i1 = jnp.min(jnp.where(logits == m1, col, E), axis=1, keepdims=True)
    oh1 = col == i1
    masked = jnp.where(oh1, -jnp.inf, logits)
    m2 = jnp.max(masked, axis=1, keepdims=True)
    i2 = jnp.min(jnp.where(masked == m2, col, E), axis=1, keepdims=True)
    oh2 = col == i2
    a1 = jax.nn.sigmoid(m1 - m2)
    gate_ref[...] = jnp.where(oh1, a1, 0.0) + jnp.where(oh2, 1.0 - a1, 0.0)


def _moe_kernel(xb_ref, gate_ref, fc1_ref, fc2_ref, out_ref, acc_ref, sem):
    e = pl.program_id(0)
    i = pl.program_id(1)

    @pl.when(jnp.logical_and(e == 0, i == 0))
    def _init():
        acc_ref[...] = jnp.zeros_like(acc_ref)

    xb = xb_ref[...]
    w1h = fc1_ref[0, 0].astype(jnp.bfloat16)
    w1g = fc1_ref[0, 1].astype(jnp.bfloat16)
    w2 = fc2_ref[0].astype(jnp.bfloat16)

    h = jax.lax.dot_general(xb, w1h, (((1,), (1,)), ((), ())),
                            preferred_element_type=jnp.float32)
    g = jax.lax.dot_general(xb, w1g, (((1,), (1,)), ((), ())),
                            preferred_element_type=jnp.float32)
    act = 0.5 * h * (1.0 + jax.lax.erf(h * 0.7071067811865476)) * (g + 1.0)

    col = jax.lax.broadcasted_iota(jnp.int32, (T, E), 1)
    gate_col = jnp.sum(jnp.where(col == e, gate_ref[...], 0.0), axis=1,
                       keepdims=True)
    act = (act * gate_col).astype(jnp.bfloat16)

    acc_ref[...] += jax.lax.dot_general(
        act, w2, (((1,), (1,)), ((), ())),
        preferred_element_type=jnp.float32)

    @pl.when(jnp.logical_and(e == E - 1, i == NI - 1))
    def _flush():
        cp = pltpu.make_async_copy(acc_ref, out_ref, sem)
        cp.start()
        cp.wait()


@jax.jit
def kernel(x, gate_w, gate_b, fc1_w, fc2_w):
    gate_dense = pl.pallas_call(
        _router_kernel,
        out_shape=jax.ShapeDtypeStruct((T, E), jnp.float32),
    )(x, gate_w, gate_b.reshape(1, E))

    xb = x.astype(jnp.bfloat16)
    fc1r = fc1_w.reshape(E, 2, INNER, H)
    return pl.pallas_call(
        _moe_kernel,
        grid=(E, NI),
        in_specs=[
            pl.BlockSpec((T, H), lambda e, i: (0, 0)),
            pl.BlockSpec((T, E), lambda e, i: (0, 0)),
            pl.BlockSpec((1, 2, TI, H), lambda e, i: (e, 0, i, 0)),
            pl.BlockSpec((1, H, TI), lambda e, i: (e, 0, i)),
        ],
        out_specs=pl.BlockSpec(memory_space=pltpu.MemorySpace.HBM),
        out_shape=jax.ShapeDtypeStruct((T, H), jnp.float32),
        scratch_shapes=[
            pltpu.VMEM((T, H), jnp.float32),
            pltpu.SemaphoreType.DMA,
        ],
        compiler_params=pltpu.CompilerParams(
            dimension_semantics=("arbitrary", "arbitrary"),
        ),
    )(xb, gate_dense, fc1r, fc2_w)
